# baseline (speedup 1.0000x reference)
import functools

import jax
import jax.numpy as jnp
from jax import lax
from jax.experimental import pallas as pl
from jax.experimental.pallas import tpu as pltpu
from jax.experimental.pallas import tpu_sc as plsc

_VOCAB = 32000
_HIDDEN = 4096
_BATCH = 2
_SEQ = 2048
_B = _BATCH * _SEQ
_NC = 2
_NS = 16
_NW = _NC * _NS
_BPW = _B // _NW
_CHUNK = 8
_NBUF = 3
_NSTEP = _BPW // _CHUNK
_G = (_NSTEP - _NBUF) // _NBUF

_mesh = plsc.VectorSubcoreMesh(core_axis_name="c", subcore_axis_name="s")


@functools.partial(
    pl.kernel,
    out_type=jax.ShapeDtypeStruct((_B, _HIDDEN), jnp.float32),
    mesh=_mesh,
    scratch_types=[
        pltpu.VMEM((_BPW,), jnp.int32),
        pltpu.VMEM((_NBUF, _CHUNK, _HIDDEN), jnp.float32),
        pltpu.SemaphoreType.DMA((_NBUF,)),
        pltpu.SemaphoreType.DMA((_NBUF,)),
    ],
)
def _embed_gather(idx_hbm, table_hbm, out_hbm, idx_v, rows_v, gsem, ssem):
    wid = lax.axis_index("s") * _NC + lax.axis_index("c")
    base = wid * _BPW
    pltpu.sync_copy(idx_hbm.at[pl.ds(base, _BPW)], idx_v)

    def g_copy(c, b):
        return pltpu.make_async_copy(
            table_hbm.at[idx_v.at[pl.ds(c * _CHUNK, _CHUNK)]],
            rows_v.at[b], gsem.at[b])

    def s_copy(c, b):
        return pltpu.make_async_copy(
            rows_v.at[b], out_hbm.at[pl.ds(base + c * _CHUNK, _CHUNK)],
            ssem.at[b])

    for b in range(_NBUF):
        g_copy(b, b).start()

    for c in (0, 1):
        g_copy(c, c % _NBUF).wait()
        s_copy(c, c % _NBUF).start()

    def outer(g, _):
        for b in range(_NBUF):
            c3 = g * _NBUF + 2 + b
            bc = (2 + b) % _NBUF
            g_copy(c3, bc).wait()
            s_copy(c3, bc).start()
            bp = b % _NBUF
            s_copy(c3 - 2, bp).wait()
            g_copy(c3 + 1, bp).start()
        return ()

    lax.fori_loop(0, 4, outer, ())

    g_copy(14, 14 % _NBUF).wait()
    s_copy(14, 14 % _NBUF).start()
    s_copy(12, 12 % _NBUF).wait()
    g_copy(15, 15 % _NBUF).start()
    g_copy(15, 15 % _NBUF).wait()
    s_copy(15, 15 % _NBUF).start()
    for c in (13, 14, 15):
        s_copy(c, c % _NBUF).wait()


def kernel(inputs, embed_weight):
    input_ids = inputs[..., 0].reshape(_B)
    attention_mask = inputs[..., 1]
    out = _embed_gather(input_ids, embed_weight)
    return out.reshape(_BATCH, _SEQ, _HIDDEN), attention_mask
